# initial kernel scaffold (unmeasured)
import jax
import jax.numpy as jnp
from jax import lax
from jax.experimental import pallas as pl
from jax.experimental.pallas import tpu as pltpu

N_DEV = 4
NB = 4


def kernel(x, w_mat, scale_x, scale_w):
    m_total, k_shard = x.shape
    k_total, n_total = w_mat.shape
    m_per = m_total // N_DEV
    nb = n_total // NB

    def body(x_ref, w_ref, sx_ref, sw_ref, out_ref,
             xg_ref, recv_ref, send_sems, recv_sems):
        j = pl.program_id(0)
        my = lax.axis_index("i")

        @pl.when(j == 0)
        def _comm():
            barrier = pltpu.get_barrier_semaphore()
            for off in range(1, N_DEV):
                pl.semaphore_signal(
                    barrier, inc=1,
                    device_id=((my + off) % N_DEV,),
                    device_id_type=pl.DeviceIdType.MESH,
                )
            pl.semaphore_wait(barrier, N_DEV - 1)

            rdmas = []
            for off in range(1, N_DEV):
                dst = (my + off) % N_DEV
                rdma = pltpu.make_async_remote_copy(
                    src_ref=x_ref.at[pl.ds(dst * m_per, m_per), :],
                    dst_ref=recv_ref.at[off - 1],
                    send_sem=send_sems.at[off - 1],
                    recv_sem=recv_sems.at[off - 1],
                    device_id=(dst,),
                    device_id_type=pl.DeviceIdType.MESH,
                )
                rdma.start()
                rdmas.append(rdma)

            xg_ref[:, pl.ds(my * k_shard, k_shard)] = (
                x_ref[pl.ds(my * m_per, m_per), :])

            for off in range(1, N_DEV):
                src = (my - off) % N_DEV
                rdmas[off - 1].wait_recv()
                xg_ref[:, pl.ds(src * k_shard, k_shard)] = recv_ref[off - 1]
            for off in range(1, N_DEV):
                rdmas[off - 1].wait_send()

        scale = sx_ref[0] * sw_ref[0]
        out_ref[...] = jnp.dot(
            xg_ref[...], w_ref[...], preferred_element_type=jnp.float32,
        ) * scale

    return pl.pallas_call(
        body,
        grid=(NB,),
        in_specs=[
            pl.BlockSpec((m_total, k_shard), lambda j: (0, 0)),
            pl.BlockSpec((k_total, nb), lambda j: (0, j)),
            pl.BlockSpec(memory_space=pltpu.SMEM),
            pl.BlockSpec(memory_space=pltpu.SMEM),
        ],
        out_specs=pl.BlockSpec((m_per, nb), lambda j: (0, j)),
        out_shape=jax.ShapeDtypeStruct((m_per, n_total), jnp.float32),
        scratch_shapes=[
            pltpu.VMEM((m_per, k_total), x.dtype),
            pltpu.VMEM((N_DEV - 1, m_per, k_shard), x.dtype),
            pltpu.SemaphoreType.DMA((N_DEV - 1,)),
            pltpu.SemaphoreType.DMA((N_DEV - 1,)),
        ],
        compiler_params=pltpu.CompilerParams(
            collective_id=0, dimension_semantics=("arbitrary",),
        ),
    )(x, w_mat, scale_x, scale_w)


# baseline (device time: 120087 ns/iter reference)
import jax
import jax.numpy as jnp
from jax import lax
from jax.experimental import pallas as pl
from jax.experimental.pallas import tpu as pltpu

N_DEV = 4
NB = 8

F8 = jnp.float8_e5m2


def kernel(x, w_mat, scale_x, scale_w):
    m_total, k_shard = x.shape
    k_total, n_total = w_mat.shape
    m_per = m_total // N_DEV
    nb = n_total // NB

    def body(x_hbm, w_ref, sx_ref, sw_ref, out_ref,
             xstage_ref, x8_ref, xg8_ref, recv_ref, w8_ref,
             copy_sem, send_sems, recv_sems):
        j = pl.program_id(0)
        my = lax.axis_index("i")

        @pl.when(j == 0)
        def _comm():
            barrier = pltpu.get_barrier_semaphore()
            for off in range(1, N_DEV):
                pl.semaphore_signal(
                    barrier, inc=1,
                    device_id=((my + off) % N_DEV,),
                    device_id_type=pl.DeviceIdType.MESH,
                )
            pl.semaphore_wait(barrier, N_DEV - 1)

            for b in range(N_DEV):
                cp = pltpu.make_async_copy(
                    x_hbm.at[pl.ds(b * m_per, m_per), :],
                    xstage_ref,
                    copy_sem,
                )
                cp.start()
                cp.wait()
                x8_ref[b] = xstage_ref[...].astype(F8)

            rdmas = []
            for off in range(1, N_DEV):
                dst = (my + off) % N_DEV
                rdma = pltpu.make_async_remote_copy(
                    src_ref=x8_ref.at[dst],
                    dst_ref=recv_ref.at[off - 1],
                    send_sem=send_sems.at[off - 1],
                    recv_sem=recv_sems.at[off - 1],
                    device_id=(dst,),
                    device_id_type=pl.DeviceIdType.MESH,
                )
                rdma.start()
                rdmas.append(rdma)

            xg8_ref[:, pl.ds(my * k_shard, k_shard)] = x8_ref[my]

            for off in range(1, N_DEV):
                src = (my - off) % N_DEV
                rdmas[off - 1].wait_recv()
                xg8_ref[:, pl.ds(src * k_shard, k_shard)] = recv_ref[off - 1]
            for off in range(1, N_DEV):
                rdmas[off - 1].wait_send()

        w8_ref[...] = w_ref[...].astype(F8)
        scale = sx_ref[0] * sw_ref[0]
        out_ref[...] = jnp.dot(
            xg8_ref[...], w8_ref[...], preferred_element_type=jnp.float32,
        ) * scale

    return pl.pallas_call(
        body,
        grid=(NB,),
        in_specs=[
            pl.BlockSpec(memory_space=pltpu.HBM),
            pl.BlockSpec((k_total, nb), lambda j: (0, j)),
            pl.BlockSpec(memory_space=pltpu.SMEM),
            pl.BlockSpec(memory_space=pltpu.SMEM),
        ],
        out_specs=pl.BlockSpec((m_per, nb), lambda j: (0, j)),
        out_shape=jax.ShapeDtypeStruct((m_per, n_total), jnp.float32),
        scratch_shapes=[
            pltpu.VMEM((m_per, k_shard), jnp.float32),
            pltpu.VMEM((N_DEV, m_per, k_shard), F8),
            pltpu.VMEM((m_per, k_total), F8),
            pltpu.VMEM((N_DEV - 1, m_per, k_shard), F8),
            pltpu.VMEM((k_total, nb), F8),
            pltpu.SemaphoreType.DMA,
            pltpu.SemaphoreType.DMA((N_DEV - 1,)),
            pltpu.SemaphoreType.DMA((N_DEV - 1,)),
        ],
        compiler_params=pltpu.CompilerParams(
            collective_id=0, dimension_semantics=("arbitrary",),
            vmem_limit_bytes=63 * 1024 * 1024,
        ),
    )(x, w_mat, scale_x, scale_w)
